# Initial kernel scaffold; baseline (speedup 1.0000x reference)
import functools

import jax
import jax.numpy as jnp
from jax import lax
from jax.experimental import pallas as pl
from jax.experimental.pallas import tpu as pltpu
from jax.experimental.pallas import tpu_sc as plsc

_N = 10000
_E = 320000
_D = 128
_HID = 128
_NCOMM = 64
_Q = 512

_W = 128
_CH = _E // _W
_NS = 16
_CH_BASE = _CH // _NS
_CH_REM = _CH % _NS
_ROWS_PER_TILE = _N // _NS

_mesh = plsc.VectorSubcoreMesh(core_axis_name="c", subcore_axis_name="s")


@functools.partial(
    pl.kernel,
    out_type=(
        jax.ShapeDtypeStruct((_N, _D), jnp.float32),
        jax.ShapeDtypeStruct((_N, _D), jnp.float32),
    ),
    mesh=_mesh,
    scratch_types=[
        pltpu.VMEM((_W,), jnp.int32),
        pltpu.VMEM((_W,), jnp.int32),
        pltpu.VMEM((_W, _D), jnp.float32),
        pltpu.VMEM_SHARED((_N, _D), jnp.float32),
        pltpu.SemaphoreType.DMA,
    ],
)
def _agg_pair(src_hbm, dst_hbm, x_hbm, q_hbm, xa_hbm, qa_hbm,
              sidx, didx, rows, acc, sem):
    c = lax.axis_index("c")
    s = lax.axis_index("s")

    @pl.when(s == 0)
    def _():
        @pl.when(c == 0)
        def _():
            pltpu.sync_copy(x_hbm, acc)

        @pl.when(c == 1)
        def _():
            pltpu.sync_copy(q_hbm, acc)

    plsc.subcore_barrier()

    n_ch = _CH_BASE + jnp.where(s < _CH_REM, 1, 0)
    lo = s * _CH_BASE + jnp.minimum(s, _CH_REM)

    def body(i, carry):
        g = lo + i
        pltpu.sync_copy(src_hbm.at[g], sidx)
        pltpu.sync_copy(dst_hbm.at[g], didx)

        @pl.when(c == 0)
        def _():
            pltpu.async_copy(x_hbm.at[sidx], rows, sem).wait()

        @pl.when(c == 1)
        def _():
            pltpu.async_copy(q_hbm.at[sidx], rows, sem).wait()

        pltpu.sync_copy(rows, acc.at[didx], add=True)
        return carry

    lax.fori_loop(0, n_ch, body, 0)

    plsc.subcore_barrier()

    r0 = s * _ROWS_PER_TILE

    @pl.when(c == 0)
    def _():
        pltpu.sync_copy(acc.at[pl.ds(r0, _ROWS_PER_TILE)],
                        xa_hbm.at[pl.ds(r0, _ROWS_PER_TILE)])

    @pl.when(c == 1)
    def _():
        pltpu.sync_copy(acc.at[pl.ds(r0, _ROWS_PER_TILE)],
                        qa_hbm.at[pl.ds(r0, _ROWS_PER_TILE)])


def _dense1_body(xa_ref, qa_ref, wb_ref, te_ref, wa_ref, ba_ref,
                 base_ref, h_ref):
    base = jnp.maximum(
        jnp.dot(xa_ref[...], wb_ref[...], preferred_element_type=jnp.float32),
        0.0) + te_ref[...]
    attr = jnp.maximum(
        jnp.dot(qa_ref[...], wa_ref[...], preferred_element_type=jnp.float32)
        + ba_ref[...], 0.0)
    base_ref[...] = base
    h_ref[...] = base + attr


def _dense2_body(ha_ref, wf_ref, bf_ref, wo_ref, bo_ref, qb_ref, wc_ref,
                 out_ref, comm_ref):
    fused = jnp.maximum(
        jnp.dot(ha_ref[...], wf_ref[...], preferred_element_type=jnp.float32)
        + bf_ref[...], 0.0)
    out_ref[...] = (
        jnp.dot(fused, wo_ref[...], preferred_element_type=jnp.float32)
        + bo_ref[...])
    comm_ref[...] = jnp.dot(qb_ref[...], wc_ref[...],
                            preferred_element_type=jnp.float32)


def kernel(x, edge_index, query_nodes, query_attrs, t,
           W_base, t_emb, W_comm,
           W_attr, b_attr, W_fuse, b_fuse, W_out, b_out):
    src = edge_index[0].astype(jnp.int32).reshape(_CH, _W)
    dst = edge_index[1].astype(jnp.int32).reshape(_CH, _W)

    xa, qa = _agg_pair(src, dst, x, query_attrs)

    base, h = pl.pallas_call(
        _dense1_body,
        out_shape=(
            jax.ShapeDtypeStruct((_N, _HID), jnp.float32),
            jax.ShapeDtypeStruct((_N, _HID), jnp.float32),
        ),
    )(xa, qa, W_base, t_emb[t].reshape(1, _HID), W_attr,
      b_attr.reshape(1, _HID))

    ha, _unused = _agg_pair(src, dst, h, h)

    qbase = base[query_nodes]

    node_pred, community_pred = pl.pallas_call(
        _dense2_body,
        out_shape=(
            jax.ShapeDtypeStruct((_N, _D), jnp.float32),
            jax.ShapeDtypeStruct((_Q, _NCOMM), jnp.float32),
        ),
    )(ha, W_fuse, b_fuse.reshape(1, _HID), W_out, b_out.reshape(1, _D),
      qbase, W_comm)

    return (node_pred, community_pred)


# speedup vs baseline: 3.6332x; 3.6332x over previous
import functools

import jax
import jax.numpy as jnp
from jax import lax
from jax.experimental import pallas as pl
from jax.experimental.pallas import tpu as pltpu
from jax.experimental.pallas import tpu_sc as plsc

_N = 10000
_E = 320000
_D = 128
_HID = 128
_NCOMM = 64
_Q = 512

_W = 128
_CH = _E // _W
_NS = 16
_CH_BASE = _CH // _NS
_CH_REM = _CH % _NS
_ROW_BLK = 624
_LAST_BLK = _N - (_NS - 1) * _ROW_BLK

_mesh = plsc.VectorSubcoreMesh(core_axis_name="c", subcore_axis_name="s")


@functools.partial(
    pl.kernel,
    out_type=jax.ShapeDtypeStruct((2 * _N, _D), jnp.float32),
    mesh=_mesh,
    scratch_types=[
        pltpu.VMEM((_W,), jnp.int32),
        pltpu.VMEM((_W,), jnp.int32),
        pltpu.VMEM((_W, _D), jnp.float32),
        pltpu.VMEM_SHARED((_N, _D), jnp.float32),
        pltpu.SemaphoreType.DMA,
    ],
)
def _agg_pair(src_hbm, dst_hbm, tab_hbm, out_hbm, sidx, didx, rows, acc, sem):
    c = lax.axis_index("c")
    s = lax.axis_index("s")
    coff = pl.multiple_of(c * _N, 8)

    @pl.when(s == 0)
    def _():
        pltpu.sync_copy(tab_hbm.at[pl.ds(coff, _N)], acc)

    plsc.subcore_barrier()

    n_ch = _CH_BASE + jnp.where(s < _CH_REM, 1, 0)
    lo = s * _CH_BASE + jnp.minimum(s, _CH_REM)
    row_off = (c * _N).astype(jnp.int32)

    def body(i, carry):
        e0 = pl.multiple_of((lo + i) * _W, 8)
        pltpu.sync_copy(src_hbm.at[pl.ds(e0, _W)], sidx)
        pltpu.sync_copy(dst_hbm.at[pl.ds(e0, _W)], didx)
        for k in range(_W // 16):
            sl = pl.ds(k * 16, 16)
            sidx[sl] = sidx[sl] + row_off
        pltpu.async_copy(tab_hbm.at[sidx], rows, sem).wait()
        pltpu.sync_copy(rows, acc.at[didx], add=True)
        return carry

    lax.fori_loop(0, n_ch, body, 0)

    plsc.subcore_barrier()

    r0 = s * _ROW_BLK

    @pl.when(s < _NS - 1)
    def _():
        pltpu.sync_copy(acc.at[pl.ds(r0, _ROW_BLK)],
                        out_hbm.at[pl.ds(coff + r0, _ROW_BLK)])

    @pl.when(s == _NS - 1)
    def _():
        pltpu.sync_copy(acc.at[pl.ds((_NS - 1) * _ROW_BLK, _LAST_BLK)],
                        out_hbm.at[pl.ds(coff + (_NS - 1) * _ROW_BLK,
                                         _LAST_BLK)])


def _dense1_body(agg_ref, wb_ref, te_ref, wa_ref, ba_ref, base_ref, h_ref):
    xa = agg_ref[0:_N, :]
    qa = agg_ref[_N:2 * _N, :]
    base = jnp.maximum(
        jnp.dot(xa, wb_ref[...], preferred_element_type=jnp.float32),
        0.0) + te_ref[...]
    attr = jnp.maximum(
        jnp.dot(qa, wa_ref[...], preferred_element_type=jnp.float32)
        + ba_ref[...], 0.0)
    base_ref[...] = base
    h_ref[...] = base + attr


def _dense2_body(agg_ref, wf_ref, bf_ref, wo_ref, bo_ref, qb_ref, wc_ref,
                 out_ref, comm_ref):
    ha = agg_ref[0:_N, :]
    fused = jnp.maximum(
        jnp.dot(ha, wf_ref[...], preferred_element_type=jnp.float32)
        + bf_ref[...], 0.0)
    out_ref[...] = (
        jnp.dot(fused, wo_ref[...], preferred_element_type=jnp.float32)
        + bo_ref[...])
    comm_ref[...] = jnp.dot(qb_ref[...], wc_ref[...],
                            preferred_element_type=jnp.float32)


def kernel(x, edge_index, query_nodes, query_attrs, t,
           W_base, t_emb, W_comm,
           W_attr, b_attr, W_fuse, b_fuse, W_out, b_out):
    src = edge_index[0].astype(jnp.int32)
    dst = edge_index[1].astype(jnp.int32)

    agg1 = _agg_pair(src, dst, jnp.concatenate([x, query_attrs], axis=0))

    base, h = pl.pallas_call(
        _dense1_body,
        out_shape=(
            jax.ShapeDtypeStruct((_N, _HID), jnp.float32),
            jax.ShapeDtypeStruct((_N, _HID), jnp.float32),
        ),
    )(agg1, W_base, t_emb[t].reshape(1, _HID), W_attr,
      b_attr.reshape(1, _HID))

    agg2 = _agg_pair(src, dst, jnp.concatenate([h, h], axis=0))

    qbase = base[query_nodes]

    node_pred, community_pred = pl.pallas_call(
        _dense2_body,
        out_shape=(
            jax.ShapeDtypeStruct((_N, _D), jnp.float32),
            jax.ShapeDtypeStruct((_Q, _NCOMM), jnp.float32),
        ),
    )(agg2, W_fuse, b_fuse.reshape(1, _HID), W_out, b_out.reshape(1, _D),
      qbase, W_comm)

    return (node_pred, community_pred)


# speedup vs baseline: 4.7289x; 1.3016x over previous
import functools

import jax
import jax.numpy as jnp
from jax import lax
from jax.experimental import pallas as pl
from jax.experimental.pallas import tpu as pltpu
from jax.experimental.pallas import tpu_sc as plsc

_N = 10000
_E = 320000
_D = 128
_HID = 128
_NCOMM = 64
_Q = 512

_W = 128
_CH = _E // _W
_NS = 16
_CH_BASE = _CH // _NS
_CH_REM = _CH % _NS
_ROW_BLK = 624
_LAST_BLK = _N - (_NS - 1) * _ROW_BLK

_mesh = plsc.VectorSubcoreMesh(core_axis_name="c", subcore_axis_name="s")

_EH = _E // 2
_CH2 = _EH // _W
_CH2_BASE = _CH2 // _NS
_CH2_REM = _CH2 % _NS


@functools.partial(
    pl.kernel,
    out_type=jax.ShapeDtypeStruct((2 * _N, _D), jnp.float32),
    mesh=_mesh,
    scratch_types=[
        pltpu.VMEM((_W,), jnp.int32),
        pltpu.VMEM((_W,), jnp.int32),
        pltpu.VMEM((_W, _D), jnp.float32),
        pltpu.VMEM_SHARED((_N, _D), jnp.float32),
        pltpu.SemaphoreType.DMA,
    ],
)
def _agg_pair(src_hbm, dst_hbm, tab_hbm, out_hbm, sidx, didx, rows, acc, sem):
    c = lax.axis_index("c")
    s = lax.axis_index("s")
    coff = pl.multiple_of(c * _N, 8)

    @pl.when(s == 0)
    def _():
        pltpu.sync_copy(tab_hbm.at[pl.ds(coff, _N)], acc)

    plsc.subcore_barrier()

    n_ch = _CH_BASE + jnp.where(s < _CH_REM, 1, 0)
    lo = s * _CH_BASE + jnp.minimum(s, _CH_REM)
    row_off = (c * _N).astype(jnp.int32)

    def body(i, carry):
        e0 = pl.multiple_of((lo + i) * _W, 8)
        pltpu.sync_copy(src_hbm.at[pl.ds(e0, _W)], sidx)
        pltpu.sync_copy(dst_hbm.at[pl.ds(e0, _W)], didx)
        for k in range(_W // 16):
            sl = pl.ds(k * 16, 16)
            sidx[sl] = sidx[sl] + row_off
        pltpu.async_copy(tab_hbm.at[sidx], rows, sem).wait()
        pltpu.sync_copy(rows, acc.at[didx], add=True)
        return carry

    lax.fori_loop(0, n_ch, body, 0)

    plsc.subcore_barrier()

    r0 = s * _ROW_BLK

    @pl.when(s < _NS - 1)
    def _():
        pltpu.sync_copy(acc.at[pl.ds(r0, _ROW_BLK)],
                        out_hbm.at[pl.ds(coff + r0, _ROW_BLK)])

    @pl.when(s == _NS - 1)
    def _():
        pltpu.sync_copy(acc.at[pl.ds((_NS - 1) * _ROW_BLK, _LAST_BLK)],
                        out_hbm.at[pl.ds(coff + (_NS - 1) * _ROW_BLK,
                                         _LAST_BLK)])


@functools.partial(
    pl.kernel,
    out_type=jax.ShapeDtypeStruct((2 * _N, _D), jnp.float32),
    mesh=_mesh,
    scratch_types=[
        pltpu.VMEM((_W,), jnp.int32),
        pltpu.VMEM((_W,), jnp.int32),
        pltpu.VMEM((_W, _D), jnp.float32),
        pltpu.VMEM_SHARED((_N, _D), jnp.float32),
        pltpu.SemaphoreType.DMA,
    ],
)
def _agg_split(src_hbm, dst_hbm, tab_hbm, out_hbm, sidx, didx, rows, acc, sem):
    c = lax.axis_index("c")
    s = lax.axis_index("s")
    coff = pl.multiple_of(c * _N, 8)

    @pl.when(s == 0)
    def _():
        pltpu.sync_copy(tab_hbm, acc)

    plsc.subcore_barrier()

    n_ch = _CH2_BASE + jnp.where(s < _CH2_REM, 1, 0)
    lo = c * _CH2 + s * _CH2_BASE + jnp.minimum(s, _CH2_REM)

    def body(i, carry):
        e0 = pl.multiple_of((lo + i) * _W, 8)
        pltpu.sync_copy(src_hbm.at[pl.ds(e0, _W)], sidx)
        pltpu.sync_copy(dst_hbm.at[pl.ds(e0, _W)], didx)
        pltpu.async_copy(tab_hbm.at[sidx], rows, sem).wait()
        pltpu.sync_copy(rows, acc.at[didx], add=True)
        return carry

    lax.fori_loop(0, n_ch, body, 0)

    plsc.subcore_barrier()

    r0 = s * _ROW_BLK

    @pl.when(s < _NS - 1)
    def _():
        pltpu.sync_copy(acc.at[pl.ds(r0, _ROW_BLK)],
                        out_hbm.at[pl.ds(coff + r0, _ROW_BLK)])

    @pl.when(s == _NS - 1)
    def _():
        pltpu.sync_copy(acc.at[pl.ds((_NS - 1) * _ROW_BLK, _LAST_BLK)],
                        out_hbm.at[pl.ds(coff + (_NS - 1) * _ROW_BLK,
                                         _LAST_BLK)])


def _dense1_body(agg_ref, wb_ref, te_ref, wa_ref, ba_ref, base_ref, h_ref):
    xa = agg_ref[0:_N, :]
    qa = agg_ref[_N:2 * _N, :]
    base = jnp.maximum(
        jnp.dot(xa, wb_ref[...], preferred_element_type=jnp.float32),
        0.0) + te_ref[...]
    attr = jnp.maximum(
        jnp.dot(qa, wa_ref[...], preferred_element_type=jnp.float32)
        + ba_ref[...], 0.0)
    base_ref[...] = base
    h_ref[...] = base + attr


def _dense2_body(agg_ref, h_ref, wf_ref, bf_ref, wo_ref, bo_ref, qb_ref,
                 wc_ref, out_ref, comm_ref):
    ha = agg_ref[0:_N, :] + agg_ref[_N:2 * _N, :] - h_ref[...]
    fused = jnp.maximum(
        jnp.dot(ha, wf_ref[...], preferred_element_type=jnp.float32)
        + bf_ref[...], 0.0)
    out_ref[...] = (
        jnp.dot(fused, wo_ref[...], preferred_element_type=jnp.float32)
        + bo_ref[...])
    comm_ref[...] = jnp.dot(qb_ref[...], wc_ref[...],
                            preferred_element_type=jnp.float32)


def kernel(x, edge_index, query_nodes, query_attrs, t,
           W_base, t_emb, W_comm,
           W_attr, b_attr, W_fuse, b_fuse, W_out, b_out):
    src = edge_index[0].astype(jnp.int32)
    dst = edge_index[1].astype(jnp.int32)

    agg1 = _agg_pair(src, dst, jnp.concatenate([x, query_attrs], axis=0))

    base, h = pl.pallas_call(
        _dense1_body,
        out_shape=(
            jax.ShapeDtypeStruct((_N, _HID), jnp.float32),
            jax.ShapeDtypeStruct((_N, _HID), jnp.float32),
        ),
    )(agg1, W_base, t_emb[t].reshape(1, _HID), W_attr,
      b_attr.reshape(1, _HID))

    agg2 = _agg_split(src, dst, h)

    qbase = base[query_nodes]

    node_pred, community_pred = pl.pallas_call(
        _dense2_body,
        out_shape=(
            jax.ShapeDtypeStruct((_N, _D), jnp.float32),
            jax.ShapeDtypeStruct((_Q, _NCOMM), jnp.float32),
        ),
    )(agg2, h, W_fuse, b_fuse.reshape(1, _HID), W_out, b_out.reshape(1, _D),
      qbase, W_comm)

    return (node_pred, community_pred)


# speedup vs baseline: 7.4925x; 1.5844x over previous
import functools

import jax
import jax.numpy as jnp
from jax import lax
from jax.experimental import pallas as pl
from jax.experimental.pallas import tpu as pltpu
from jax.experimental.pallas import tpu_sc as plsc

_N = 10000
_E = 320000
_D = 128
_HID = 128
_NCOMM = 64
_Q = 512

_W = 128
_CH = _E // _W
_NS = 16
_CH_BASE = _CH // _NS
_CH_REM = _CH % _NS
_ROW_BLK = 624
_LAST_BLK = _N - (_NS - 1) * _ROW_BLK

_mesh = plsc.VectorSubcoreMesh(core_axis_name="c", subcore_axis_name="s")

_EH = _E // 2
_CH2 = _EH // _W
_CH2_BASE = _CH2 // _NS
_CH2_REM = _CH2 % _NS

_SCRATCH = [
    pltpu.VMEM((_W,), jnp.int32),
    pltpu.VMEM((_W,), jnp.int32),
    pltpu.VMEM((_W, _D), jnp.float32),
    pltpu.VMEM((_W,), jnp.int32),
    pltpu.VMEM((_W,), jnp.int32),
    pltpu.VMEM((_W, _D), jnp.float32),
    pltpu.VMEM_SHARED((_N, _D), jnp.float32),
    pltpu.SemaphoreType.DMA,
    pltpu.SemaphoreType.DMA,
]


def _pipe_loop(src_hbm, dst_hbm, tab_hbm, acc,
               s0, d0, r0, m0, s1, d1, r1, m1, lo, n_ch, srcbase):
    def fire(sidx, didx, rows, sem, g):
        e0 = pl.multiple_of(g * _W, 8)
        pltpu.sync_copy(src_hbm.at[pl.ds(srcbase + e0, _W)], sidx)
        pltpu.sync_copy(dst_hbm.at[pl.ds(e0, _W)], didx)
        pltpu.async_copy(tab_hbm.at[sidx], rows, sem)

    def drain(sidx, didx, rows, sem):
        pltpu.make_async_copy(tab_hbm.at[sidx], rows, sem).wait()
        pltpu.sync_copy(rows, acc.at[didx], add=True)

    fire(s0, d0, r0, m0, lo)
    fire(s1, d1, r1, m1, lo + 1)

    n_pairs = (n_ch + 1) // 2

    def body(j, carry):
        g2 = lo + 2 * j + 2
        drain(s0, d0, r0, m0)

        @pl.when(2 * j + 2 < n_ch)
        def _():
            fire(s0, d0, r0, m0, g2)

        @pl.when(2 * j + 1 < n_ch)
        def _():
            drain(s1, d1, r1, m1)

        @pl.when(2 * j + 3 < n_ch)
        def _():
            fire(s1, d1, r1, m1, g2 + 1)

        return carry

    lax.fori_loop(0, n_pairs, body, 0)


def _write_out(acc, out_hbm, s, coff):
    r0 = s * _ROW_BLK

    @pl.when(s < _NS - 1)
    def _():
        pltpu.sync_copy(acc.at[pl.ds(r0, _ROW_BLK)],
                        out_hbm.at[pl.ds(coff + r0, _ROW_BLK)])

    @pl.when(s == _NS - 1)
    def _():
        pltpu.sync_copy(acc.at[pl.ds((_NS - 1) * _ROW_BLK, _LAST_BLK)],
                        out_hbm.at[pl.ds(coff + (_NS - 1) * _ROW_BLK,
                                         _LAST_BLK)])


@functools.partial(
    pl.kernel,
    out_type=jax.ShapeDtypeStruct((2 * _N, _D), jnp.float32),
    mesh=_mesh,
    scratch_types=_SCRATCH,
)
def _agg_pair(src2_hbm, dst_hbm, tab_hbm, out_hbm,
              s0, d0, r0, s1, d1, r1, acc, m0, m1):
    c = lax.axis_index("c")
    s = lax.axis_index("s")
    coff = pl.multiple_of(c * _N, 8)

    @pl.when(s == 0)
    def _():
        pltpu.sync_copy(tab_hbm.at[pl.ds(coff, _N)], acc)

    plsc.subcore_barrier()

    n_ch = _CH_BASE + jnp.where(s < _CH_REM, 1, 0)
    lo = s * _CH_BASE + jnp.minimum(s, _CH_REM)
    srcbase = pl.multiple_of(c * _E, 8)

    _pipe_loop(src2_hbm, dst_hbm, tab_hbm, acc,
               s0, d0, r0, m0, s1, d1, r1, m1, lo, n_ch, srcbase)

    plsc.subcore_barrier()
    _write_out(acc, out_hbm, s, coff)


@functools.partial(
    pl.kernel,
    out_type=jax.ShapeDtypeStruct((2 * _N, _D), jnp.float32),
    mesh=_mesh,
    scratch_types=_SCRATCH,
)
def _agg_split(src_hbm, dst_hbm, tab_hbm, out_hbm,
               s0, d0, r0, s1, d1, r1, acc, m0, m1):
    c = lax.axis_index("c")
    s = lax.axis_index("s")
    coff = pl.multiple_of(c * _N, 8)

    @pl.when(s == 0)
    def _():
        pltpu.sync_copy(tab_hbm, acc)

    plsc.subcore_barrier()

    n_ch = _CH2_BASE + jnp.where(s < _CH2_REM, 1, 0)
    lo = c * _CH2 + s * _CH2_BASE + jnp.minimum(s, _CH2_REM)

    _pipe_loop(src_hbm, dst_hbm, tab_hbm, acc,
               s0, d0, r0, m0, s1, d1, r1, m1, lo, n_ch, 0)

    plsc.subcore_barrier()
    _write_out(acc, out_hbm, s, coff)


def _dense1_body(agg_ref, wb_ref, te_ref, wa_ref, ba_ref, base_ref, h_ref):
    xa = agg_ref[0:_N, :]
    qa = agg_ref[_N:2 * _N, :]
    base = jnp.maximum(
        jnp.dot(xa, wb_ref[...], preferred_element_type=jnp.float32),
        0.0) + te_ref[...]
    attr = jnp.maximum(
        jnp.dot(qa, wa_ref[...], preferred_element_type=jnp.float32)
        + ba_ref[...], 0.0)
    base_ref[...] = base
    h_ref[...] = base + attr


def _dense2_body(agg_ref, h_ref, wf_ref, bf_ref, wo_ref, bo_ref, qb_ref,
                 wc_ref, out_ref, comm_ref):
    ha = agg_ref[0:_N, :] + agg_ref[_N:2 * _N, :] - h_ref[...]
    fused = jnp.maximum(
        jnp.dot(ha, wf_ref[...], preferred_element_type=jnp.float32)
        + bf_ref[...], 0.0)
    out_ref[...] = (
        jnp.dot(fused, wo_ref[...], preferred_element_type=jnp.float32)
        + bo_ref[...])
    comm_ref[...] = jnp.dot(qb_ref[...], wc_ref[...],
                            preferred_element_type=jnp.float32)


def kernel(x, edge_index, query_nodes, query_attrs, t,
           W_base, t_emb, W_comm,
           W_attr, b_attr, W_fuse, b_fuse, W_out, b_out):
    src = edge_index[0].astype(jnp.int32)
    dst = edge_index[1].astype(jnp.int32)
    src2 = jnp.concatenate([src, src + _N])

    agg1 = _agg_pair(src2, dst, jnp.concatenate([x, query_attrs], axis=0))

    base, h = pl.pallas_call(
        _dense1_body,
        out_shape=(
            jax.ShapeDtypeStruct((_N, _HID), jnp.float32),
            jax.ShapeDtypeStruct((_N, _HID), jnp.float32),
        ),
    )(agg1, W_base, t_emb[t].reshape(1, _HID), W_attr,
      b_attr.reshape(1, _HID))

    agg2 = _agg_split(src, dst, h)

    qbase = base[query_nodes]

    node_pred, community_pred = pl.pallas_call(
        _dense2_body,
        out_shape=(
            jax.ShapeDtypeStruct((_N, _D), jnp.float32),
            jax.ShapeDtypeStruct((_Q, _NCOMM), jnp.float32),
        ),
    )(agg2, h, W_fuse, b_fuse.reshape(1, _HID), W_out, b_out.reshape(1, _D),
      qbase, W_comm)

    return (node_pred, community_pred)


# speedup vs baseline: 9.4735x; 1.2644x over previous
import functools

import jax
import jax.numpy as jnp
from jax import lax
from jax.experimental import pallas as pl
from jax.experimental.pallas import tpu as pltpu
from jax.experimental.pallas import tpu_sc as plsc

_N = 10000
_E = 320000
_D = 128
_HID = 128
_NCOMM = 64
_Q = 512

_W = 128
_CH = _E // _W
_NS = 16
_CH_BASE = _CH // _NS
_CH_REM = _CH % _NS
_ROW_BLK = 624
_LAST_BLK = _N - (_NS - 1) * _ROW_BLK

_mesh = plsc.VectorSubcoreMesh(core_axis_name="c", subcore_axis_name="s")

_EH = _E // 2
_CH2 = _EH // _W
_CH2_BASE = _CH2 // _NS
_CH2_REM = _CH2 % _NS


_KB = 64

_SCRATCH = [
    pltpu.VMEM((_KB, 2, _W), jnp.int32),
    pltpu.VMEM((_W, _D), jnp.float32),
    pltpu.VMEM((_W, _D), jnp.float32),
    pltpu.VMEM_SHARED((_N, _D), jnp.float32),
    pltpu.SemaphoreType.DMA,
    pltpu.SemaphoreType.DMA,
]


def _block_loop(idx_hbm, tab_hbm, acc, islab, r0, m0, r1, m1, lo, n_ch):
    n_blk = (n_ch + _KB - 1) // _KB

    def blk_body(b, carry):
        pltpu.sync_copy(idx_hbm.at[pl.ds(lo + b * _KB, _KB)], islab)
        nb = jnp.minimum(n_ch - b * _KB, _KB)
        _pipe_loop(tab_hbm, acc, islab, r0, m0, r1, m1, nb)
        return carry

    lax.fori_loop(0, n_blk, blk_body, 0)


def _pipe_loop(tab_hbm, acc, islab, r0, m0, r1, m1, n_ch):
    def fire(rows, sem, j):
        pltpu.async_copy(tab_hbm.at[islab.at[j, 0]], rows, sem)

    def drain(rows, sem, j):
        pltpu.make_async_copy(tab_hbm.at[islab.at[j, 0]], rows, sem).wait()
        pltpu.sync_copy(rows, acc.at[islab.at[j, 1]], add=True)

    fire(r0, m0, 0)
    fire(r1, m1, 1)

    n_pairs = (n_ch + 1) // 2

    def body(j, carry):
        drain(r0, m0, 2 * j)

        @pl.when(2 * j + 2 < n_ch)
        def _():
            fire(r0, m0, 2 * j + 2)

        @pl.when(2 * j + 1 < n_ch)
        def _():
            drain(r1, m1, 2 * j + 1)

        @pl.when(2 * j + 3 < n_ch)
        def _():
            fire(r1, m1, 2 * j + 3)

        return carry

    lax.fori_loop(0, n_pairs, body, 0)


def _write_out(acc, out_hbm, s, coff):
    r0 = s * _ROW_BLK

    @pl.when(s < _NS - 1)
    def _():
        pltpu.sync_copy(acc.at[pl.ds(r0, _ROW_BLK)],
                        out_hbm.at[pl.ds(coff + r0, _ROW_BLK)])

    @pl.when(s == _NS - 1)
    def _():
        pltpu.sync_copy(acc.at[pl.ds((_NS - 1) * _ROW_BLK, _LAST_BLK)],
                        out_hbm.at[pl.ds(coff + (_NS - 1) * _ROW_BLK,
                                         _LAST_BLK)])


@functools.partial(
    pl.kernel,
    out_type=jax.ShapeDtypeStruct((2 * _N, _D), jnp.float32),
    mesh=_mesh,
    scratch_types=_SCRATCH,
)
def _agg_pair(idx_hbm, tab_hbm, out_hbm,
              islab, r0, r1, acc, m0, m1):
    c = lax.axis_index("c")
    s = lax.axis_index("s")
    coff = pl.multiple_of(c * _N, 8)

    @pl.when(s == 0)
    def _():
        pltpu.sync_copy(tab_hbm.at[pl.ds(coff, _N)], acc)

    plsc.subcore_barrier()

    n_ch = _CH_BASE + jnp.where(s < _CH_REM, 1, 0)
    lo = c * _CH + s * _CH_BASE + jnp.minimum(s, _CH_REM)

    _block_loop(idx_hbm, tab_hbm, acc, islab, r0, m0, r1, m1, lo, n_ch)

    plsc.subcore_barrier()
    _write_out(acc, out_hbm, s, coff)


@functools.partial(
    pl.kernel,
    out_type=jax.ShapeDtypeStruct((2 * _N, _D), jnp.float32),
    mesh=_mesh,
    scratch_types=_SCRATCH,
)
def _agg_split(idx_hbm, tab_hbm, out_hbm,
               islab, r0, r1, acc, m0, m1):
    c = lax.axis_index("c")
    s = lax.axis_index("s")
    coff = pl.multiple_of(c * _N, 8)

    @pl.when(s == 0)
    def _():
        pltpu.sync_copy(tab_hbm, acc)

    plsc.subcore_barrier()

    n_ch = _CH2_BASE + jnp.where(s < _CH2_REM, 1, 0)
    lo = c * _CH2 + s * _CH2_BASE + jnp.minimum(s, _CH2_REM)

    _block_loop(idx_hbm, tab_hbm, acc, islab, r0, m0, r1, m1, lo, n_ch)

    plsc.subcore_barrier()
    _write_out(acc, out_hbm, s, coff)


def _dense1_body(agg_ref, wb_ref, te_ref, wa_ref, ba_ref, base_ref, h_ref):
    xa = agg_ref[0:_N, :]
    qa = agg_ref[_N:2 * _N, :]
    base = jnp.maximum(
        jnp.dot(xa, wb_ref[...], preferred_element_type=jnp.float32),
        0.0) + te_ref[...]
    attr = jnp.maximum(
        jnp.dot(qa, wa_ref[...], preferred_element_type=jnp.float32)
        + ba_ref[...], 0.0)
    base_ref[...] = base
    h_ref[...] = base + attr


def _dense2_body(agg_ref, h_ref, wf_ref, bf_ref, wo_ref, bo_ref, qb_ref,
                 wc_ref, out_ref, comm_ref):
    ha = agg_ref[0:_N, :] + agg_ref[_N:2 * _N, :] - h_ref[...]
    fused = jnp.maximum(
        jnp.dot(ha, wf_ref[...], preferred_element_type=jnp.float32)
        + bf_ref[...], 0.0)
    out_ref[...] = (
        jnp.dot(fused, wo_ref[...], preferred_element_type=jnp.float32)
        + bo_ref[...])
    comm_ref[...] = jnp.dot(qb_ref[...], wc_ref[...],
                            preferred_element_type=jnp.float32)


def kernel(x, edge_index, query_nodes, query_attrs, t,
           W_base, t_emb, W_comm,
           W_attr, b_attr, W_fuse, b_fuse, W_out, b_out):
    src = edge_index[0].astype(jnp.int32)
    dst = edge_index[1].astype(jnp.int32)

    srcc = src.reshape(_CH, _W)
    dstc = dst.reshape(_CH, _W)
    pad = jnp.zeros((_KB, 2, _W), jnp.int32)
    idx_pair = jnp.concatenate([
        jnp.stack([jnp.concatenate([srcc.reshape(1, _CH, _W),
                                    (srcc + _N).reshape(1, _CH, _W)], axis=0),
                   jnp.broadcast_to(dstc, (2, _CH, _W))],
                  axis=2).reshape(2 * _CH, 2, _W),
        pad], axis=0)
    idx_split = jnp.concatenate(
        [jnp.stack([srcc, dstc], axis=1), pad], axis=0)

    agg1 = _agg_pair(idx_pair, jnp.concatenate([x, query_attrs], axis=0))

    base, h = pl.pallas_call(
        _dense1_body,
        out_shape=(
            jax.ShapeDtypeStruct((_N, _HID), jnp.float32),
            jax.ShapeDtypeStruct((_N, _HID), jnp.float32),
        ),
    )(agg1, W_base, t_emb[t].reshape(1, _HID), W_attr,
      b_attr.reshape(1, _HID))

    agg2 = _agg_split(idx_split, h)

    qbase = base[query_nodes]

    node_pred, community_pred = pl.pallas_call(
        _dense2_body,
        out_shape=(
            jax.ShapeDtypeStruct((_N, _D), jnp.float32),
            jax.ShapeDtypeStruct((_Q, _NCOMM), jnp.float32),
        ),
    )(agg2, h, W_fuse, b_fuse.reshape(1, _HID), W_out, b_out.reshape(1, _D),
      qbase, W_comm)

    return (node_pred, community_pred)
